# Initial kernel scaffold; baseline (speedup 1.0000x reference)
import jax
import jax.numpy as jnp
from jax.experimental import pallas as pl


def kernel(z, pos, emb_W, emb_b, mlp_w1, mlp_b1, mlp_w2, mlp_b2, cf_w1, cf_w2, cf_b2, blk_w, blk_b, out1_W, out1_b, out2_W, out2_b, batch_dimensions):
    raise NotImplementedError("write your pallas kernel here")


# speedup vs baseline: 5.6544x; 5.6544x over previous
import functools
import math

import jax
import jax.numpy as jnp
from jax import lax
from jax.experimental import pallas as pl
from jax.experimental.pallas import tpu as pltpu

_N = 16384
_A = 32
_B = _N // _A
_DIM = 32
_NF = 32
_NG = 10
_NI = 2
_CUTOFF = 5.0
_GAMMA = 4.0
_LOG2 = math.log(2.0)

_MB = 4


def _ssp(x):
    return jnp.maximum(x, 0.0) + jnp.log1p(jnp.exp(-jnp.abs(x))) - _LOG2


def _schnet_kernel(z_ref, pi_ref, pj_ref, mu_ref, odiag_ref,
                   emb_W_ref, emb_b_ref,
                   mlp_w1_ref, mlp_b1_ref, mlp_w2_ref, mlp_b2_ref,
                   cf_w1_ref, cf_w2_ref, cf_b2_ref, blk_w_ref, blk_b_ref,
                   out1_W_ref, out1_b_ref, out2_W_ref, out2_b_ref,
                   out_ref, *, mb):
    f32 = jnp.float32
    zv = z_ref[...]
    h = zv * emb_W_ref[0:1, :] + emb_b_ref[0:1, :]
    h2 = h.reshape(mb * _A, _DIM)

    d2 = None
    for c in range(3):
        pic = pi_ref[:, :, :, c:c + 1]
        pjc = pj_ref[:, :, :, c:c + 1]
        dif = pic - pjc
        d2 = dif * dif if d2 is None else d2 + dif * dif
    d = jnp.sqrt(d2 + 1e-12)

    mu = mu_ref[0:1, :]
    rbf = jnp.exp(-_GAMMA * (d - mu) ** 2)
    rbf2 = rbf.reshape(mb * _A * _A, _NG)

    cosw = 0.5 * (jnp.cos(d * (math.pi / _CUTOFF)) + 1.0)
    cosw = jnp.where(d <= _CUTOFF, cosw, 0.0)
    cm = cosw * odiag_ref[...]

    for t in range(_NI):
        wf = _ssp(rbf2 @ mlp_w1_ref[t] + mlp_b1_ref[t:t + 1, :])
        wf = wf @ mlp_w2_ref[t] + mlp_b2_ref[t:t + 1, :]
        wf4 = wf.reshape(mb, _A, _A, _NF) * cm
        xl = h2 @ cf_w1_ref[t]
        xlb = xl.reshape(mb, 1, _A, _NF)
        agg = jnp.sum(wf4 * xlb, axis=2)
        x2 = agg.reshape(mb * _A, _NF) @ cf_w2_ref[t] + cf_b2_ref[t:t + 1, :]
        x2 = _ssp(x2)
        x2 = x2 @ blk_w_ref[t] + blk_b_ref[t:t + 1, :]
        h2 = h2 + x2

    o = _ssp(h2 @ out1_W_ref[...] + out1_b_ref[0:1, :])
    o = o @ out2_W_ref[...] + out2_b_ref[0:1, :]
    out_ref[...] = jnp.sum(o.reshape(mb, _A, 1), axis=1).reshape(1, mb, 1)


def kernel(z, pos, emb_W, emb_b, mlp_w1, mlp_b1, mlp_w2, mlp_b2, cf_w1,
           cf_w2, cf_b2, blk_w, blk_b, out1_W, out1_b, out2_W, out2_b,
           batch_dimensions):
    mb = _MB
    grid = (_B // mb,)
    z3 = z.reshape(_B, _A, 1)
    pos_i = pos.reshape(_B, _A, 1, 3)
    pos_j = pos.reshape(_B, 1, _A, 3)
    mu_v = jnp.linspace(0.0, _CUTOFF, _NG, dtype=jnp.float32).reshape(1, _NG)
    odiag = (1.0 - jnp.eye(_A, dtype=jnp.float32)).reshape(1, _A, _A, 1)
    emb_b2 = emb_b.reshape(1, _DIM)
    out1_b2 = out1_b.reshape(1, _DIM // 2)
    out2_b2 = out2_b.reshape(1, 1)

    def tile(shape):
        n = len(shape)
        return pl.BlockSpec(shape, lambda m, _n=n: (0,) * _n)

    in_specs = [
        pl.BlockSpec((mb, _A, 1), lambda m: (m, 0, 0)),
        pl.BlockSpec((mb, _A, 1, 3), lambda m: (m, 0, 0, 0)),
        pl.BlockSpec((mb, 1, _A, 3), lambda m: (m, 0, 0, 0)),
        tile((1, _NG)), tile((1, _A, _A, 1)),
        tile(emb_W.shape), tile(emb_b2.shape),
        tile(mlp_w1.shape), tile(mlp_b1.shape),
        tile(mlp_w2.shape), tile(mlp_b2.shape),
        tile(cf_w1.shape), tile(cf_w2.shape), tile(cf_b2.shape),
        tile(blk_w.shape), tile(blk_b.shape),
        tile(out1_W.shape), tile(out1_b2.shape),
        tile(out2_W.shape), tile(out2_b2.shape),
    ]
    out = pl.pallas_call(
        functools.partial(_schnet_kernel, mb=mb),
        grid=grid,
        in_specs=in_specs,
        out_specs=pl.BlockSpec((1, mb, 1), lambda m: (m, 0, 0)),
        out_shape=jax.ShapeDtypeStruct((_B // mb, mb, 1), jnp.float32),
        compiler_params=pltpu.CompilerParams(
            dimension_semantics=("arbitrary",)),
    )(z3, pos_i, pos_j, mu_v, odiag, emb_W, emb_b2, mlp_w1, mlp_b1, mlp_w2, mlp_b2,
      cf_w1, cf_w2, cf_b2, blk_w, blk_b, out1_W, out1_b2, out2_W, out2_b2)
    return out.reshape(_B, 1)
